# Initial kernel scaffold; baseline (speedup 1.0000x reference)
import jax
import jax.numpy as jnp
from jax.experimental import pallas as pl


def kernel(x, W_enc, W_dec, b_enc, b_dec):
    raise NotImplementedError("write your pallas kernel here")


# speedup vs baseline: 7.8434x; 7.8434x over previous
import functools

import jax
import jax.numpy as jnp
from jax.experimental import pallas as pl
from jax.experimental.pallas import tpu as pltpu

_K = 32


def _key_to_f32(k):
    b = jnp.where(k >= 0, k, k ^ jnp.int32(0x7FFFFFFF))
    return jax.lax.bitcast_convert_type(b, jnp.float32)


def _body(x_ref, we_ref, wd_ref, be_ref, bd_ref, xh_ref, z_ref,
          pre_ref, acc_ref, thr_ref, *, T, C, NC):
    p = pl.program_id(1)
    c = pl.program_id(2)

    @pl.when(p == 0)
    def _encode():
        xm = x_ref[...] - bd_ref[...]
        pre_ref[:, pl.ds(c * C, C)] = (
            jnp.dot(xm, we_ref[...], preferred_element_type=jnp.float32)
            + be_ref[...]
        )

    @pl.when((p == 1) & (c == 0))
    def _threshold():
        ii = jnp.iinfo(jnp.int32)
        lo0 = jnp.full((T, 1), ii.min, jnp.int32)
        hi0 = jnp.full((T, 1), ii.max, jnp.int32)

        def step(_, lh):
            lo, hi = lh
            mid = (lo >> 1) + (hi >> 1) + (lo & hi & 1)
            mf = _key_to_f32(mid)
            cnt = jnp.sum((pre_ref[...] >= mf).astype(jnp.float32),
                          axis=1, keepdims=True)
            ge = cnt >= float(_K)
            return jnp.where(ge, mid, lo), jnp.where(ge, hi, mid)

        lo, _ = jax.lax.fori_loop(0, 32, step, (lo0, hi0))
        thr_ref[...] = _key_to_f32(lo)

    @pl.when(p == 1)
    def _select_decode():
        pre_c = pre_ref[:, pl.ds(c * C, C)]
        zc = jnp.where(pre_c >= thr_ref[...], pre_c, 0.0)
        z_ref[...] = zc
        part = jnp.dot(zc.astype(jnp.bfloat16), wd_ref[...],
                       preferred_element_type=jnp.float32)

        @pl.when(c == 0)
        def _():
            acc_ref[...] = part

        @pl.when(c > 0)
        def _():
            acc_ref[...] += part

        @pl.when(c == NC - 1)
        def _():
            xh_ref[...] = acc_ref[...] + bd_ref[...]


@jax.jit
def kernel(x, W_enc, W_dec, b_enc, b_dec):
    N, D = x.shape
    S = W_enc.shape[1]
    T = 512 if N % 512 == 0 else N
    C = 1024 if S % 1024 == 0 else S
    NT, NC = N // T, S // C

    wd_b = W_dec.astype(jnp.bfloat16)
    be2 = b_enc.reshape(1, S)
    bd2 = b_dec.reshape(1, D)

    grid = (NT, 2, NC)
    last = NC - 1

    x_hat, z = pl.pallas_call(
        functools.partial(_body, T=T, C=C, NC=NC),
        grid=grid,
        in_specs=[
            pl.BlockSpec((T, D), lambda t, p, c: (t, 0)),
            pl.BlockSpec((D, C), lambda t, p, c: (0, jnp.where(p == 0, c, last))),
            pl.BlockSpec((C, D), lambda t, p, c: (jnp.where(p == 1, c, 0), 0)),
            pl.BlockSpec((1, C), lambda t, p, c: (0, jnp.where(p == 0, c, last))),
            pl.BlockSpec((1, D), lambda t, p, c: (0, 0)),
        ],
        out_specs=[
            pl.BlockSpec((T, D), lambda t, p, c: (t, 0)),
            pl.BlockSpec((T, C), lambda t, p, c: (t, jnp.where(p == 1, c, 0))),
        ],
        out_shape=[
            jax.ShapeDtypeStruct((N, D), jnp.float32),
            jax.ShapeDtypeStruct((N, S), jnp.float32),
        ],
        scratch_shapes=[
            pltpu.VMEM((T, S), jnp.float32),
            pltpu.VMEM((T, D), jnp.float32),
            pltpu.VMEM((T, 1), jnp.float32),
        ],
        compiler_params=pltpu.CompilerParams(
            dimension_semantics=("arbitrary", "arbitrary", "arbitrary"),
        ),
    )(x, W_enc, wd_b, be2, bd2)
    return (x_hat, z)


# speedup vs baseline: 8.4336x; 1.0753x over previous
import functools

import jax
import jax.numpy as jnp
from jax.experimental import pallas as pl
from jax.experimental.pallas import tpu as pltpu

_K = 32


def _key_to_f32(k):
    b = jnp.where(k >= 0, k, k ^ jnp.int32(0x7FFFFFFF))
    return jax.lax.bitcast_convert_type(b, jnp.float32)


def _f32_to_key(f):
    b = jax.lax.bitcast_convert_type(f, jnp.int32)
    return jnp.where(b >= 0, b, b ^ jnp.int32(0x7FFFFFFF))


def _body(x_ref, we_ref, wd_ref, be_ref, bd_ref, xh_ref, z_ref,
          pre_ref, acc_ref, thr_ref, *, T, C, NC):
    S = C * NC
    p = pl.program_id(1)
    c = pl.program_id(2)

    @pl.when(p == 0)
    def _encode():
        xm = x_ref[...] - bd_ref[...]
        pre_ref[:, pl.ds(c * C, C)] = (
            jnp.dot(xm, we_ref[...], preferred_element_type=jnp.float32)
            + be_ref[...]
        )

    @pl.when((p == 1) & (c == 0))
    def _threshold():
        G = min(1024, S)
        gm = pre_ref[:, pl.ds(0, G)]
        for j in range(1, S // G):
            gm = jnp.maximum(gm, pre_ref[:, pl.ds(j * G, G)])
        rowmax = jnp.max(gm, axis=1, keepdims=True)

        def extract(i, g):
            m = jnp.max(g, axis=1, keepdims=True)
            return jnp.where(g >= m, -jnp.inf, g)

        gm = jax.lax.fori_loop(0, _K - 1, extract, gm)
        gm32 = jnp.max(gm, axis=1, keepdims=True)

        lo0 = _f32_to_key(gm32)
        hi0 = _f32_to_key(rowmax) + 1

        def cond(lh):
            lo, hi = lh
            return jnp.max(hi - lo) > 1

        def step(lh):
            lo, hi = lh
            mid = (lo >> 1) + (hi >> 1) + (lo & hi & 1)
            mf = _key_to_f32(mid)
            cnt = jnp.sum((pre_ref[...] >= mf).astype(jnp.float32),
                          axis=1, keepdims=True)
            ge = cnt >= float(_K)
            return jnp.where(ge, mid, lo), jnp.where(ge, hi, mid)

        lo, _ = jax.lax.while_loop(cond, step, (lo0, hi0))
        thr_ref[...] = _key_to_f32(lo)

    @pl.when(p == 1)
    def _select_decode():
        pre_c = pre_ref[:, pl.ds(c * C, C)]
        zc = jnp.where(pre_c >= thr_ref[...], pre_c, 0.0)
        z_ref[...] = zc
        part = jnp.dot(zc.astype(jnp.bfloat16), wd_ref[...],
                       preferred_element_type=jnp.float32)

        @pl.when(c == 0)
        def _():
            acc_ref[...] = part

        @pl.when(c > 0)
        def _():
            acc_ref[...] += part

        @pl.when(c == NC - 1)
        def _():
            xh_ref[...] = acc_ref[...] + bd_ref[...]


@jax.jit
def kernel(x, W_enc, W_dec, b_enc, b_dec):
    N, D = x.shape
    S = W_enc.shape[1]
    T = 512 if N % 512 == 0 else N
    C = 1024 if S % 1024 == 0 else S
    NT, NC = N // T, S // C

    wd_b = W_dec.astype(jnp.bfloat16)
    be2 = b_enc.reshape(1, S)
    bd2 = b_dec.reshape(1, D)

    grid = (NT, 2, NC)
    last = NC - 1

    x_hat, z = pl.pallas_call(
        functools.partial(_body, T=T, C=C, NC=NC),
        grid=grid,
        in_specs=[
            pl.BlockSpec((T, D), lambda t, p, c: (t, 0)),
            pl.BlockSpec((D, C), lambda t, p, c: (0, jnp.where(p == 0, c, last))),
            pl.BlockSpec((C, D), lambda t, p, c: (jnp.where(p == 1, c, 0), 0)),
            pl.BlockSpec((1, C), lambda t, p, c: (0, jnp.where(p == 0, c, last))),
            pl.BlockSpec((1, D), lambda t, p, c: (0, 0)),
        ],
        out_specs=[
            pl.BlockSpec((T, D), lambda t, p, c: (t, 0)),
            pl.BlockSpec((T, C), lambda t, p, c: (t, jnp.where(p == 1, c, 0))),
        ],
        out_shape=[
            jax.ShapeDtypeStruct((N, D), jnp.float32),
            jax.ShapeDtypeStruct((N, S), jnp.float32),
        ],
        scratch_shapes=[
            pltpu.VMEM((T, S), jnp.float32),
            pltpu.VMEM((T, D), jnp.float32),
            pltpu.VMEM((T, 1), jnp.float32),
        ],
        compiler_params=pltpu.CompilerParams(
            dimension_semantics=("arbitrary", "arbitrary", "arbitrary"),
        ),
    )(x, W_enc, wd_b, be2, bd2)
    return (x_hat, z)
